# Initial kernel scaffold; baseline (speedup 1.0000x reference)
import jax
import jax.numpy as jnp
from jax.experimental import pallas as pl


def kernel(x, pos, params):
    raise NotImplementedError("write your pallas kernel here")


# speedup vs baseline: 1.0257x; 1.0257x over previous
import math
import jax
import jax.numpy as jnp
from jax.experimental import pallas as pl
from jax.experimental.pallas import tpu as pltpu

_EPS = 1e-5


def _bn_rows(t, g, b):
    m = jnp.mean(t, axis=0, keepdims=True)
    v = jnp.mean(t * t, axis=0, keepdims=True) - m * m
    return (t - m) / jnp.sqrt(v + _EPS) * g + b


def _bn_groups(h, K, C, g, b):
    s1 = jnp.mean(h, axis=0, keepdims=True)
    s2 = jnp.mean(h * h, axis=0, keepdims=True)
    mc = sum(s1[:, k * C:(k + 1) * C] for k in range(K)) / K
    e2 = sum(s2[:, k * C:(k + 1) * C] for k in range(K)) / K
    vc = e2 - mc * mc
    mt = jnp.concatenate([mc] * K, axis=1)
    vt = jnp.concatenate([vc] * K, axis=1)
    return (h - mt) / jnp.sqrt(vt + _EPS) * g + b


def _elu(x):
    return jnp.where(x > 0, x, jnp.exp(jnp.minimum(x, 0.0)) - 1.0)


def _dot(a, b):
    return jnp.dot(a, b, preferred_element_type=jnp.float32)


_VMEM = pltpu.CompilerParams(vmem_limit_bytes=120 * 1024 * 1024)


def _blk(M):
    return 2048 if M % 2048 == 0 else M


def _stats(x):
    M, C = x.shape
    bm = _blk(M)

    def body(x_ref, o_ref):
        i = pl.program_id(0)
        xv = x_ref[:]
        blk = jnp.concatenate([jnp.sum(xv, axis=0, keepdims=True),
                               jnp.sum(xv * xv, axis=0, keepdims=True)], axis=0)

        @pl.when(i == 0)
        def _():
            o_ref[:] = blk

        @pl.when(i > 0)
        def _():
            o_ref[:] = o_ref[:] + blk

    return pl.pallas_call(
        body, grid=(M // bm,),
        in_specs=[pl.BlockSpec((bm, C), lambda i: (i, 0))],
        out_specs=pl.BlockSpec((2, C), lambda i: (0, 0)),
        out_shape=jax.ShapeDtypeStruct((2, C), jnp.float32),
        compiler_params=_VMEM)(x)


def _apply_bn(x, st, Mtot, g, b):
    m = st[0:1, :] / Mtot
    v = st[1:2, :] / Mtot - m * m
    return (x - m) / jnp.sqrt(v + _EPS) * g + b


def _apply_gbn(x, st, Mtot, K, C, g, b):
    s1, s2 = st[0:1, :], st[1:2, :]
    mc = sum(s1[:, k * C:(k + 1) * C] for k in range(K)) / (Mtot * K)
    e2 = sum(s2[:, k * C:(k + 1) * C] for k in range(K)) / (Mtot * K)
    vc = e2 - mc * mc
    mt = jnp.concatenate([mc] * K, axis=1)
    vt = jnp.concatenate([vc] * K, axis=1)
    return (x - mt) / jnp.sqrt(vt + _EPS) * g + b


def _xconv_call(pf, xn, pp, M, K, D, Cin, Cd, Cout):
    Cm = Cin + Cd
    (w1T, b1, g1, be1, w2T, b2, g2, be2, wlT, bl, ga, ba,
     c1T, cb1, gb, bb, c2T, cb2, gc, bc, dwT, db, fwT, fb) = pp
    bm = _blk(M)
    nb = M // bm

    def full(a):
        return pl.BlockSpec(a.shape, lambda i: tuple(0 for _ in a.shape))

    def row(shape):
        return pl.BlockSpec(shape, lambda i: (i, 0))

    def g1_body(pf_ref, w1T_ref, b1_ref, o_ref):
        pf_v = pf_ref[:]
        w1 = w1T_ref[:]
        a = jnp.concatenate(
            [_dot(pf_v[:, k * D:(k + 1) * D], w1) for k in range(K)], axis=1)
        o_ref[:] = _elu(a + b1_ref[:])

    a1 = pl.pallas_call(
        g1_body, grid=(nb,),
        in_specs=[row((bm, K * D)), full(w1T), full(b1)],
        out_specs=row((bm, K * Cd)),
        out_shape=jax.ShapeDtypeStruct((M, K * Cd), jnp.float32),
        compiler_params=_VMEM)(pf, w1T, b1)

    st1 = _stats(a1)

    def g2_body(a1_ref, st1_ref, g1_ref, be1_ref, w2T_ref, b2_ref, o_ref):
        h1 = _apply_gbn(a1_ref[:], st1_ref[:], M, K, Cd, g1_ref[:], be1_ref[:])
        w2 = w2T_ref[:]
        a = jnp.concatenate(
            [_dot(h1[:, k * Cd:(k + 1) * Cd], w2) for k in range(K)], axis=1)
        o_ref[:] = _elu(a + b2_ref[:])

    a2 = pl.pallas_call(
        g2_body, grid=(nb,),
        in_specs=[row((bm, K * Cd)), full(st1), full(g1), full(be1),
                  full(w2T), full(b2)],
        out_specs=row((bm, K * Cd)),
        out_shape=jax.ShapeDtypeStruct((M, K * Cd), jnp.float32),
        compiler_params=_VMEM)(a1, st1, g1, be1, w2T, b2)

    st2 = _stats(a2)

    def t0_body(pf_ref, wlT_ref, bl_ref, o_ref):
        o_ref[:] = _elu(_dot(pf_ref[:], wlT_ref[:]) + bl_ref[:])

    t0 = pl.pallas_call(
        t0_body, grid=(nb,),
        in_specs=[row((bm, K * D)), full(wlT), full(bl)],
        out_specs=row((bm, K * K)),
        out_shape=jax.ShapeDtypeStruct((M, K * K), jnp.float32),
        compiler_params=_VMEM)(pf, wlT, bl)

    sta = _stats(t0)

    def _tstage(tin, st, gg, bb_, cT, cb, last):
        def body(t_ref, st_ref, g_ref, b_ref, c_ref, cb_ref, o_ref):
            t = _apply_bn(t_ref[:], st_ref[:], M, g_ref[:], b_ref[:])
            c = c_ref[:]
            t = jnp.concatenate(
                [_dot(t[:, g * K:(g + 1) * K], c[g]) for g in range(K)], axis=1)
            t = t + cb_ref[:]
            o_ref[:] = t if last else _elu(t)

        return pl.pallas_call(
            body, grid=(nb,),
            in_specs=[row((bm, K * K)), full(st), full(gg), full(bb_),
                      full(cT), full(cb)],
            out_specs=row((bm, K * K)),
            out_shape=jax.ShapeDtypeStruct((M, K * K), jnp.float32),
            compiler_params=_VMEM)(tin, st, gg, bb_, cT, cb)

    t1 = _tstage(t0, sta, ga, ba, c1T, cb1, last=False)
    stb = _stats(t1)
    t2 = _tstage(t1, stb, gb, bb, c2T, cb2, last=True)
    stc = _stats(t2)

    def c_body(a2_ref, st2_ref, g2_ref, be2_ref, xn_ref, t_ref,
               stc_ref, gc_ref, bc_ref,
               dwT_ref, db_ref, fwT_ref, fb_ref, o_ref):
        hv = _apply_gbn(a2_ref[:], st2_ref[:], M, K, Cd, g2_ref[:], be2_ref[:])
        tv = _apply_bn(t_ref[:], stc_ref[:], M, gc_ref[:], bc_ref[:])
        xv, dw = xn_ref[:], dwT_ref[:]
        acc = jnp.zeros((bm, Cm), jnp.float32)
        for k in range(K):
            u = _dot(tv[:, k * K:(k + 1) * K], dw)
            xsk = jnp.concatenate(
                [hv[:, k * Cd:(k + 1) * Cd], xv[:, k * Cin:(k + 1) * Cin]], axis=1)
            acc = acc + xsk * u
        acc = acc + db_ref[:]
        o_ref[:] = _dot(acc, fwT_ref[:]) + fb_ref[:]

    return pl.pallas_call(
        c_body,
        grid=(nb,),
        in_specs=[row((bm, K * Cd)), full(st2), full(g2), full(be2),
                  row((bm, K * Cin)), row((bm, K * K)),
                  full(stc), full(gc), full(bc),
                  full(dwT), full(db), full(fwT), full(fb)],
        out_specs=row((bm, Cout)),
        out_shape=jax.ShapeDtypeStruct((M, Cout), jnp.float32),
        compiler_params=_VMEM)(a2, st2, g2, be2, xn, t2, stc, gc, bc,
                               dwT, db, fwT, fb)


def _prep(p, K):
    row = lambda a: a.reshape(1, -1)
    tile = lambda a: jnp.concatenate([a.reshape(1, -1)] * K, axis=1)
    return (
        p['mlp1_w1'].T, tile(p['mlp1_b1']), tile(p['mlp1_g1']), tile(p['mlp1_be1']),
        p['mlp1_w2'].T, tile(p['mlp1_b2']), tile(p['mlp1_g2']), tile(p['mlp1_be2']),
        p['mlp2_wl'].T, row(p['mlp2_bl']), row(p['mlp2_ga']), row(p['mlp2_ba']),
        p['mlp2_cw1'].transpose(0, 2, 1), row(p['mlp2_cb1']),
        row(p['mlp2_gb']), row(p['mlp2_bb']),
        p['mlp2_cw2'].transpose(0, 2, 1), row(p['mlp2_cb2']),
        row(p['mlp2_gc']), row(p['mlp2_bc']),
        p['conv_dw'][:, 0, :].T, row(p['conv_db']),
        p['conv_fw'].T, row(p['conv_fb']),
    )


def _knn(pos, k):
    sq = jnp.sum(pos * pos, axis=-1)
    d = sq[:, :, None] + sq[:, None, :] - 2.0 * jnp.einsum('bnd,bmd->bnm', pos, pos)
    _, idx = jax.lax.top_k(-d, k)
    return idx


def _gather(val, idx):
    return jax.vmap(lambda v, i: v[i])(val, idx)


def _fps(pos, ratio):
    B, n, _ = pos.shape
    m = int(math.ceil(ratio * n))
    dist = jnp.full((B, n), jnp.inf)
    cur = jnp.zeros((B,), dtype=jnp.int32)
    ar = jnp.arange(B)
    idxs = []
    for _ in range(m):
        idxs.append(cur)
        psel = pos[ar, cur]
        d = jnp.sum((pos - psel[:, None, :]) ** 2, axis=-1)
        dist = jnp.minimum(dist, d)
        cur = jnp.argmax(dist, axis=1).astype(jnp.int32)
    return jnp.stack(idxs, axis=1)


def _xconv(x, pos, p, K, dil, key):
    B, n, D = pos.shape
    Cin = x.shape[-1]
    Cd = p['mlp1_w1'].shape[0]
    Cout = p['conv_fw'].shape[0]
    assert int(math.ceil(Cout / (Cin + Cd))) == 1
    idx = _knn(pos, K * dil)
    if dil > 1:
        sel = jax.random.randint(key, (B, n, K), 0, K * dil)
        idx = jnp.take_along_axis(idx, sel, axis=2)
    nbr_pos = _gather(pos, idx)
    rel = nbr_pos - pos[:, :, None, :]
    M = B * n
    pf = rel.reshape(M, K * D)
    xn = _gather(x, idx).reshape(M, K * Cin)
    out = _xconv_call(pf, xn, _prep(p, K), M, K, D, Cin, Cd, Cout)
    return out.reshape(B, n, Cout)


def _relu_bn(x, g, b):
    B, n, C = x.shape

    def body(x_ref, g_ref, b_ref, o_ref):
        o_ref[:] = jnp.maximum(_bn_rows(x_ref[:], g_ref[:], b_ref[:]), 0.0)

    out = pl.pallas_call(
        body, out_shape=jax.ShapeDtypeStruct((B * n, C), jnp.float32))(
            x.reshape(B * n, C), g.reshape(1, -1), b.reshape(1, -1))
    return out.reshape(B, n, C)


def _tail(x, g, b, l1w, l1b, l2w, l2b):
    B, n, C = x.shape
    Co = l2w.shape[0]

    def body(x_ref, g_ref, b_ref, w1_ref, b1_ref, w2_ref, b2_ref, o_ref):
        xv = x_ref[:]
        s = sum(xv[:, k * C:(k + 1) * C] for k in range(n)) / n
        h = _bn_rows(s, g_ref[:], b_ref[:])
        h = jnp.maximum(h, 0.0)
        h = _dot(h, w1_ref[:]) + b1_ref[:]
        o_ref[:] = _dot(h, w2_ref[:]) + b2_ref[:]

    return pl.pallas_call(
        body, out_shape=jax.ShapeDtypeStruct((B, Co), jnp.float32))(
            x.reshape(B, n * C), g.reshape(1, -1), b.reshape(1, -1),
            l1w.T, l1b.reshape(1, -1), l2w.T, l2b.reshape(1, -1))


def kernel(x, pos, params):
    key = jax.random.key(42)
    x = _xconv(x, pos, params['cv1'], 8, 1, key)
    idx = _fps(pos, 0.33)
    x, pos = _gather(x, idx), _gather(pos, idx)
    x = _relu_bn(x, params['bn1_g'], params['bn1_b'])
    x = _xconv(x, pos, params['cv2'], 8, 2, jax.random.fold_in(key, 2))
    x = _relu_bn(x, params['bn2_g'], params['bn2_b'])
    idx = _fps(pos, 0.33)
    x, pos = _gather(x, idx), _gather(pos, idx)
    x = _xconv(x, pos, params['cv3'], 12, 2, jax.random.fold_in(key, 3))
    x = _relu_bn(x, params['bn3_g'], params['bn3_b'])
    idx = _fps(pos, 0.33)
    x, pos = _gather(x, idx), _gather(pos, idx)
    x = _xconv(x, pos, params['cv4'], 16, 2, jax.random.fold_in(key, 4))
    x = _relu_bn(x, params['bn4_g'], params['bn4_b'])
    x = _xconv(x, pos, params['cv5'], 16, 2, jax.random.fold_in(key, 5))
    return _tail(x, params['bn5_g'], params['bn5_b'],
                 params['lin1_w'], params['lin1_b'],
                 params['lin2_w'], params['lin2_b'])
